# baseline (speedup 1.0000x reference)
import jax
import jax.numpy as jnp
from jax.experimental import pallas as pl
from jax.experimental.pallas import tpu as pltpu

_B, _C, _H, _W = 8, 3, 512, 512
_D, _K = 64, 3
_OH, _OW = _H - _K + 1, _W - _K + 1
_TR = 56
_NR = (_OH + _TR - 1) // _TR
_HALO = 8
_NHB = _H // _HALO - 1


def _conv_body(xa_ref, xb_ref, w_ref, b_ref, o_ref):
    slabs = []
    for c in range(_C):
        v = jnp.concatenate([xa_ref[0, c], xb_ref[0, c]], axis=0)
        for dy in range(_K):
            rows = v[dy:dy + _TR, :]
            for dx in range(_K):
                s = pltpu.roll(rows, _W - dx, 1) if dx else rows
                slabs.append(s.astype(jnp.bfloat16))
    slabs.append(jnp.ones((_TR, _W), jnp.bfloat16))
    patch = jnp.stack(slabs, axis=0)
    bsum = jnp.sum(b_ref[...], axis=1, keepdims=True)
    w_aug = jnp.concatenate(
        [w_ref[...], bsum], axis=1).astype(jnp.bfloat16)
    o_ref[0] = jnp.einsum(
        "dk,ktj->dtj", w_aug, patch,
        preferred_element_type=jnp.float32,
    ).astype(jnp.bfloat16)


def kernel(x, filters, bias):
    w2 = filters.reshape(_D, _C * _K * _K)
    b2 = bias.reshape(_D, _C * _K * _K)
    staged = pl.pallas_call(
        _conv_body,
        grid=(_B, _NR),
        in_specs=[
            pl.BlockSpec((1, _C, _TR, _W), lambda b, i: (b, 0, i, 0)),
            pl.BlockSpec(
                (1, _C, _HALO, _W),
                lambda b, i: (
                    b, 0,
                    jnp.minimum((i + 1) * (_TR // _HALO), _NHB), 0)),
            pl.BlockSpec((_D, _C * _K * _K), lambda b, i: (0, 0)),
            pl.BlockSpec((_D, _C * _K * _K), lambda b, i: (0, 0)),
        ],
        out_specs=pl.BlockSpec((1, _D, _TR, _W), lambda b, i: (b, 0, i, 0)),
        out_shape=jax.ShapeDtypeStruct((_B, _D, _OH, _W), jnp.bfloat16),
        compiler_params=pltpu.CompilerParams(
            dimension_semantics=("parallel", "arbitrary"),
        ),
    )(x, x, w2, b2)
    return staged[:, :, :, :_OW].astype(jnp.float32)


# speedup vs baseline: 1.5367x; 1.5367x over previous
import jax
import jax.numpy as jnp
from jax.experimental import pallas as pl
from jax.experimental.pallas import tpu as pltpu

_B, _C, _H, _W = 8, 3, 512, 512
_D, _K = 64, 3
_OH, _OW = _H - _K + 1, _W - _K + 1
_TW = 128
_NW = (_OW + _TW - 1) // _TW


def _conv_body(xa_ref, xb_ref, w_ref, b_ref, o_ref):
    slabs = []
    for c in range(_C):
        full = jnp.concatenate([xa_ref[0, c], xb_ref[0, c]], axis=1)
        for dy in range(_K):
            for dx in range(_K):
                slabs.append(full[dy:dy + _OH, dx:dx + _TW])
    patch = jnp.stack(slabs, axis=0)
    patch = jnp.concatenate(
        [patch, jnp.ones((1, _OH, _TW), jnp.float32)], axis=0)
    bsum = jnp.sum(b_ref[...], axis=1, keepdims=True)
    w_aug = jnp.concatenate([w_ref[...], bsum], axis=1)
    o_ref[0] = jnp.einsum(
        "dk,ktj->dtj", w_aug, patch,
        preferred_element_type=jnp.float32,
    )


def kernel(x, filters, bias):
    w2 = filters.reshape(_D, _C * _K * _K)
    b2 = bias.reshape(_D, _C * _K * _K)
    return pl.pallas_call(
        _conv_body,
        grid=(_B, _NW),
        in_specs=[
            pl.BlockSpec((1, _C, _H, _TW), lambda b, j: (b, 0, 0, j)),
            pl.BlockSpec(
                (1, _C, _H, _TW),
                lambda b, j: (b, 0, 0, jnp.minimum(j + 1, _NW - 1))),
            pl.BlockSpec((_D, _C * _K * _K), lambda b, j: (0, 0)),
            pl.BlockSpec((_D, _C * _K * _K), lambda b, j: (0, 0)),
        ],
        out_specs=pl.BlockSpec((1, _D, _OH, _TW), lambda b, j: (b, 0, 0, j)),
        out_shape=jax.ShapeDtypeStruct((_B, _D, _OH, _OW), jnp.float32),
        compiler_params=pltpu.CompilerParams(
            dimension_semantics=("parallel", "arbitrary"),
        ),
    )(x, x, w2, b2)
